# baseline (device time: 15441 ns/iter reference)
import jax
import jax.numpy as jnp
from jax import lax
from jax.experimental import pallas as pl
from jax.experimental.pallas import tpu as pltpu

N_DEV = 4


def kernel(x, Wq, K_ext, V_ext, Wo):
    B, Sql, E = x.shape
    _, Skl, Hq, Dh = K_ext.shape
    HD = Hq * Dh
    Skv = N_DEV * Skl

    def body(x_ref, wq_ref, k_ref, v_ref, wo_ref, out_ref,
             kfull, vfull, ksend, krecv, vsend, vrecv):
        my = lax.axis_index("i")
        left = lax.rem(my + (N_DEV - 1), N_DEV)
        right = lax.rem(my + 1, N_DEV)
        diag = lax.rem(my + 2, N_DEV)
        peers = (left, right, diag)

        barrier = pltpu.get_barrier_semaphore()
        for nbr in peers:
            pltpu.semaphore_signal(
                barrier, inc=1,
                device_id=(nbr,), device_id_type=pl.DeviceIdType.MESH,
            )
        pltpu.semaphore_wait(barrier, len(peers))

        kfull[:, pl.ds(my * Skl, Skl), :] = (
            k_ref[...].astype(jnp.bfloat16).reshape(B, Skl, HD))
        vfull[:, pl.ds(my * Skl, Skl), :] = (
            v_ref[...].astype(jnp.bfloat16).reshape(B, Skl, HD))

        sends = []
        for j, dest in enumerate(peers):
            for buf, ssem, rsem in ((kfull, ksend, krecv),
                                    (vfull, vsend, vrecv)):
                r = pltpu.make_async_remote_copy(
                    src_ref=buf.at[:, pl.ds(my * Skl, Skl), :],
                    dst_ref=buf.at[:, pl.ds(my * Skl, Skl), :],
                    send_sem=ssem.at[j], recv_sem=rsem.at[j],
                    device_id=(dest,), device_id_type=pl.DeviceIdType.MESH,
                )
                r.start()
                sends.append(r)

        wq = wq_ref[...].astype(jnp.bfloat16)
        wo = wo_ref[...].astype(jnp.bfloat16)

        q = [
            (jnp.dot(x_ref[b].astype(jnp.bfloat16), wq,
                     preferred_element_type=jnp.float32)
             * 0.125).astype(jnp.bfloat16)
            for b in range(B)
        ]

        qrow = lax.broadcasted_iota(jnp.int32, (Sql, Skl), 0) + my * Sql
        kcol = lax.broadcasted_iota(jnp.int32, (Sql, Skl), 1)

        acc = [[jnp.zeros((Sql, Dh), jnp.float32) for _ in range(Hq)]
               for _ in range(B)]
        lsum = [[jnp.zeros((Sql, 1), jnp.float32) for _ in range(Hq)]
                for _ in range(B)]

        def process_block(origin):
            ki = kcol + origin * Skl
            mask = (jnp.abs(qrow - ki) <= 128) | (ki < 32) | (qrow < 32)
            for b in range(B):
                kb = kfull[b, pl.ds(origin * Skl, Skl), :]
                vb = vfull[b, pl.ds(origin * Skl, Skl), :]
                for h in range(Hq):
                    qh = q[b][:, h * Dh:(h + 1) * Dh]
                    kh = kb[:, h * Dh:(h + 1) * Dh]
                    s = lax.dot_general(
                        qh, kh, (((1,), (1,)), ((), ())),
                        preferred_element_type=jnp.float32,
                    )
                    p = jnp.exp(jnp.where(mask, s, -1e9))
                    lsum[b][h] = lsum[b][h] + jnp.sum(p, axis=-1,
                                                      keepdims=True)
                    acc[b][h] = acc[b][h] + jnp.dot(
                        p.astype(jnp.bfloat16), vb[:, h * Dh:(h + 1) * Dh],
                        preferred_element_type=jnp.float32,
                    )

        process_block(my)
        for j, origin in enumerate((right, left, diag)):
            recv_k = pltpu.make_async_remote_copy(
                src_ref=kfull.at[:, pl.ds(origin * Skl, Skl), :],
                dst_ref=kfull.at[:, pl.ds(origin * Skl, Skl), :],
                send_sem=ksend.at[j], recv_sem=krecv.at[j],
                device_id=(origin,), device_id_type=pl.DeviceIdType.MESH,
            )
            recv_v = pltpu.make_async_remote_copy(
                src_ref=vfull.at[:, pl.ds(origin * Skl, Skl), :],
                dst_ref=vfull.at[:, pl.ds(origin * Skl, Skl), :],
                send_sem=vsend.at[j], recv_sem=vrecv.at[j],
                device_id=(origin,), device_id_type=pl.DeviceIdType.MESH,
            )
            recv_k.wait_recv()
            recv_v.wait_recv()
            process_block(origin)

        for b in range(B):
            ctx = jnp.concatenate(
                [(acc[b][h] / lsum[b][h]).astype(jnp.bfloat16)
                 for h in range(Hq)],
                axis=1,
            )
            out_ref[b] = jnp.dot(ctx, wo,
                                 preferred_element_type=jnp.float32)

        for r in sends:
            r.wait_send()

    return pl.pallas_call(
        body,
        out_shape=jax.ShapeDtypeStruct((B, Sql, E), jnp.float32),
        in_specs=[pl.BlockSpec(memory_space=pltpu.VMEM)] * 5,
        out_specs=pl.BlockSpec(memory_space=pltpu.VMEM),
        scratch_shapes=[
            pltpu.VMEM((B, Skv, HD), jnp.bfloat16),
            pltpu.VMEM((B, Skv, HD), jnp.bfloat16),
            pltpu.SemaphoreType.DMA((N_DEV - 1,)),
            pltpu.SemaphoreType.DMA((N_DEV - 1,)),
            pltpu.SemaphoreType.DMA((N_DEV - 1,)),
            pltpu.SemaphoreType.DMA((N_DEV - 1,)),
        ],
        compiler_params=pltpu.CompilerParams(collective_id=0),
    )(x, Wq, K_ext, V_ext, Wo)


# device time: 15217 ns/iter; 1.0147x vs baseline; 1.0147x over previous
import jax
import jax.numpy as jnp
from jax import lax
from jax.experimental import pallas as pl
from jax.experimental.pallas import tpu as pltpu

N_DEV = 4


def kernel(x, Wq, K_ext, V_ext, Wo):
    B, Sql, E = x.shape
    _, Skl, Hq, Dh = K_ext.shape
    HD = Hq * Dh
    Skv = N_DEV * Skl

    def body(x_ref, wq_ref, k_ref, v_ref, wo_ref, out_ref,
             kfull, vfull, kst, vst, ksend, krecv, vsend, vrecv, insem):
        my = lax.axis_index("i")
        left = lax.rem(my + (N_DEV - 1), N_DEV)
        right = lax.rem(my + 1, N_DEV)
        diag = lax.rem(my + 2, N_DEV)
        peers = (left, right, diag)

        cp_k = pltpu.make_async_copy(k_ref, kst, insem.at[0])
        cp_v = pltpu.make_async_copy(v_ref, vst, insem.at[1])
        cp_k.start()
        cp_v.start()

        barrier = pltpu.get_barrier_semaphore()
        for nbr in peers:
            pltpu.semaphore_signal(
                barrier, inc=1,
                device_id=(nbr,), device_id_type=pl.DeviceIdType.MESH,
            )

        cp_k.wait()
        kfull[:, pl.ds(my * Skl, Skl), :] = (
            kst[...].astype(jnp.bfloat16).reshape(B, Skl, HD))
        cp_v.wait()
        vfull[:, pl.ds(my * Skl, Skl), :] = (
            vst[...].astype(jnp.bfloat16).reshape(B, Skl, HD))

        pltpu.semaphore_wait(barrier, len(peers))

        sends = []
        for j, dest in enumerate(peers):
            for buf, ssem, rsem in ((kfull, ksend, krecv),
                                    (vfull, vsend, vrecv)):
                r = pltpu.make_async_remote_copy(
                    src_ref=buf.at[:, pl.ds(my * Skl, Skl), :],
                    dst_ref=buf.at[:, pl.ds(my * Skl, Skl), :],
                    send_sem=ssem.at[j], recv_sem=rsem.at[j],
                    device_id=(dest,), device_id_type=pl.DeviceIdType.MESH,
                )
                r.start()
                sends.append(r)

        wq = wq_ref[...].astype(jnp.bfloat16)
        wo = wo_ref[...].astype(jnp.bfloat16)

        q = [
            (jnp.dot(x_ref[b].astype(jnp.bfloat16), wq,
                     preferred_element_type=jnp.float32)
             * 0.125).astype(jnp.bfloat16)
            for b in range(B)
        ]

        qrow = lax.broadcasted_iota(jnp.int32, (Sql, Skl), 0) + my * Sql
        kcol = lax.broadcasted_iota(jnp.int32, (Sql, Skl), 1)

        acc = [[jnp.zeros((Sql, Dh), jnp.float32) for _ in range(Hq)]
               for _ in range(B)]
        lsum = [[jnp.zeros((Sql, 1), jnp.float32) for _ in range(Hq)]
                for _ in range(B)]

        def process_block(origin):
            ki = kcol + origin * Skl
            mask = (jnp.abs(qrow - ki) <= 128) | (ki < 32) | (qrow < 32)
            for b in range(B):
                kb = kfull[b, pl.ds(origin * Skl, Skl), :]
                vb = vfull[b, pl.ds(origin * Skl, Skl), :]
                for h in range(Hq):
                    qh = q[b][:, h * Dh:(h + 1) * Dh]
                    kh = kb[:, h * Dh:(h + 1) * Dh]
                    s = lax.dot_general(
                        qh, kh, (((1,), (1,)), ((), ())),
                        preferred_element_type=jnp.float32,
                    )
                    p = jnp.exp(jnp.where(mask, s, -1e9))
                    lsum[b][h] = lsum[b][h] + jnp.sum(p, axis=-1,
                                                      keepdims=True)
                    acc[b][h] = acc[b][h] + jnp.dot(
                        p.astype(jnp.bfloat16), vb[:, h * Dh:(h + 1) * Dh],
                        preferred_element_type=jnp.float32,
                    )

        process_block(my)
        for j, origin in enumerate((right, left, diag)):
            recv_k = pltpu.make_async_remote_copy(
                src_ref=kfull.at[:, pl.ds(origin * Skl, Skl), :],
                dst_ref=kfull.at[:, pl.ds(origin * Skl, Skl), :],
                send_sem=ksend.at[j], recv_sem=krecv.at[j],
                device_id=(origin,), device_id_type=pl.DeviceIdType.MESH,
            )
            recv_v = pltpu.make_async_remote_copy(
                src_ref=vfull.at[:, pl.ds(origin * Skl, Skl), :],
                dst_ref=vfull.at[:, pl.ds(origin * Skl, Skl), :],
                send_sem=vsend.at[j], recv_sem=vrecv.at[j],
                device_id=(origin,), device_id_type=pl.DeviceIdType.MESH,
            )
            recv_k.wait_recv()
            recv_v.wait_recv()
            process_block(origin)

        for b in range(B):
            ctx = jnp.concatenate(
                [(acc[b][h] / lsum[b][h]).astype(jnp.bfloat16)
                 for h in range(Hq)],
                axis=1,
            )
            out_ref[b] = jnp.dot(ctx, wo,
                                 preferred_element_type=jnp.float32)

        for r in sends:
            r.wait_send()

    return pl.pallas_call(
        body,
        out_shape=jax.ShapeDtypeStruct((B, Sql, E), jnp.float32),
        in_specs=[
            pl.BlockSpec(memory_space=pltpu.VMEM),
            pl.BlockSpec(memory_space=pltpu.VMEM),
            pl.BlockSpec(memory_space=pl.ANY),
            pl.BlockSpec(memory_space=pl.ANY),
            pl.BlockSpec(memory_space=pltpu.VMEM),
        ],
        out_specs=pl.BlockSpec(memory_space=pltpu.VMEM),
        scratch_shapes=[
            pltpu.VMEM((B, Skv, HD), jnp.bfloat16),
            pltpu.VMEM((B, Skv, HD), jnp.bfloat16),
            pltpu.VMEM((B, Skl, Hq, Dh), jnp.float32),
            pltpu.VMEM((B, Skl, Hq, Dh), jnp.float32),
            pltpu.SemaphoreType.DMA((N_DEV - 1,)),
            pltpu.SemaphoreType.DMA((N_DEV - 1,)),
            pltpu.SemaphoreType.DMA((N_DEV - 1,)),
            pltpu.SemaphoreType.DMA((N_DEV - 1,)),
            pltpu.SemaphoreType.DMA((2,)),
        ],
        compiler_params=pltpu.CompilerParams(collective_id=0),
    )(x, Wq, K_ext, V_ext, Wo)
